# baseline (device time: 15696 ns/iter reference)
import jax
import jax.numpy as jnp
from jax import lax
from jax.experimental import pallas as pl
from jax.experimental.pallas import tpu as pltpu

N_DEV = 8
E_LOCAL = 2


def kernel(x, router_W, route_idx, expert_W, shared_W):
    T, D = x.shape
    _, _, H = expert_W.shape
    E = N_DEV * E_LOCAL

    def body(x_ref, rw_ref, idx_ref, ew_ref, sw_ref, out_ref,
             sendbuf_ref, comm_ref, send_sems, recv_sems):
        my = lax.axis_index("i")

        barrier_sem = pltpu.get_barrier_semaphore()
        for k in range(1, N_DEV):
            pl.semaphore_signal(
                barrier_sem, inc=1,
                device_id=(lax.rem(my + k, N_DEV),),
                device_id_type=pl.DeviceIdType.MESH,
            )
        pl.semaphore_wait(barrier_sem, N_DEV - 1)

        ew_bf = ew_ref[...].astype(jnp.bfloat16).reshape(E_LOCAL * D, H)
        sendbuf_ref[...] = ew_bf
        for k in range(1, N_DEV):
            rdma = pltpu.make_async_remote_copy(
                src_ref=sendbuf_ref,
                dst_ref=comm_ref.at[k - 1],
                send_sem=send_sems.at[k - 1],
                recv_sem=recv_sems.at[k - 1],
                device_id=(lax.rem(my + k, N_DEV),),
                device_id_type=pl.DeviceIdType.MESH,
            )
            rdma.start()

        out_ref[...] = jnp.zeros((T, H), jnp.float32)
        for k in range(1, N_DEV):
            recv = pltpu.make_async_remote_copy(
                src_ref=sendbuf_ref,
                dst_ref=comm_ref.at[k - 1],
                send_sem=send_sems.at[k - 1],
                recv_sem=recv_sems.at[k - 1],
                device_id=(0,),
                device_id_type=pl.DeviceIdType.MESH,
            )
            recv.wait_recv()
        if True:
            pass

        for k in range(1, N_DEV):
            send = pltpu.make_async_remote_copy(
                src_ref=sendbuf_ref,
                dst_ref=comm_ref.at[k - 1],
                send_sem=send_sems.at[k - 1],
                recv_sem=recv_sems.at[k - 1],
                device_id=(0,),
                device_id_type=pl.DeviceIdType.MESH,
            )
            send.wait_send()

    return pl.pallas_call(
        body,
        out_shape=jax.ShapeDtypeStruct((T, H), jnp.float32),
        in_specs=[pl.BlockSpec(memory_space=pltpu.VMEM)] * 5,
        out_specs=pl.BlockSpec(memory_space=pltpu.VMEM),
        scratch_shapes=[
            pltpu.VMEM((E_LOCAL * D, H), jnp.bfloat16),
            pltpu.VMEM((N_DEV - 1, E_LOCAL * D, H), jnp.bfloat16),
            pltpu.SemaphoreType.DMA((N_DEV - 1,)),
            pltpu.SemaphoreType.DMA((N_DEV - 1,)),
        ],
        compiler_params=pltpu.CompilerParams(collective_id=0),
    )(x, router_W, route_idx, expert_W, shared_W)


# device time: 14022 ns/iter; 1.1194x vs baseline; 1.1194x over previous
import jax
import jax.numpy as jnp
from jax import lax
from jax.experimental import pallas as pl
from jax.experimental.pallas import tpu as pltpu

N_DEV = 8
E_LOCAL = 2


def kernel(x, router_W, route_idx, expert_W, shared_W):
    T, D = x.shape
    _, _, H = expert_W.shape
    E = N_DEV * E_LOCAL

    def body(x_ref, rw_ref, idx_ref, ew_ref, sw_ref, out_ref,
             sendq_ref, sends_ref, commq_ref, comms_ref,
             send_sems, recv_sems):
        my = lax.axis_index("i")

        barrier_sem = pltpu.get_barrier_semaphore()
        for k in range(1, N_DEV):
            pl.semaphore_signal(
                barrier_sem, inc=1,
                device_id=(lax.rem(my + k, N_DEV),),
                device_id_type=pl.DeviceIdType.MESH,
            )
        pl.semaphore_wait(barrier_sem, N_DEV - 1)

        for j in range(E_LOCAL):
            wj = ew_ref[j]
            mj = jnp.max(jnp.abs(wj), axis=0, keepdims=True)
            sinv = jnp.where(mj > 0, 127.0 / mj, 0.0)
            sendq_ref[j * D:(j + 1) * D, :] = jnp.round(
                wj * sinv).astype(jnp.int8)
            sends_ref[j:j + 1, :] = mj * (1.0 / 127.0)

        for k in range(1, N_DEV):
            tgt = lax.rem(my + k, N_DEV)
            qd = pltpu.make_async_remote_copy(
                src_ref=sendq_ref,
                dst_ref=commq_ref.at[k - 1],
                send_sem=send_sems.at[k - 1],
                recv_sem=recv_sems.at[k - 1],
                device_id=(tgt,),
                device_id_type=pl.DeviceIdType.MESH,
            )
            qd.start()
            sd = pltpu.make_async_remote_copy(
                src_ref=sends_ref,
                dst_ref=comms_ref.at[k - 1],
                send_sem=send_sems.at[N_DEV - 1 + k - 1],
                recv_sem=recv_sems.at[N_DEV - 1 + k - 1],
                device_id=(tgt,),
                device_id_type=pl.DeviceIdType.MESH,
            )
            sd.start()

        x_bf = x_ref[...].astype(jnp.bfloat16)
        scores = jnp.dot(x_ref[...], rw_ref[...],
                         preferred_element_type=jnp.float32)
        s_max = jnp.max(scores, axis=-1, keepdims=True)
        p = jnp.exp(scores - s_max)
        probs = p / jnp.sum(p, axis=-1, keepdims=True)
        eidx = lax.broadcasted_iota(jnp.int32, (T, E), 1)
        coef = jnp.where(idx_ref[...] == eidx, probs, 0.0)
        coef_rot = pltpu.roll(
            coef.astype(jnp.bfloat16),
            jnp.mod(E - E_LOCAL * my, E), 1)

        acc = jnp.dot(x_bf, sw_ref[...].astype(jnp.bfloat16),
                      preferred_element_type=jnp.float32)

        ew_bf = ew_ref[...].astype(jnp.bfloat16)
        for j in range(E_LOCAL):
            acc = acc + jnp.dot(x_bf * coef_rot[:, j:j + 1], ew_bf[j],
                                preferred_element_type=jnp.float32)

        for k in range(1, N_DEV):
            for off in (0, N_DEV - 1):
                recv = pltpu.make_async_remote_copy(
                    src_ref=sendq_ref if off == 0 else sends_ref,
                    dst_ref=(commq_ref if off == 0 else comms_ref).at[k - 1],
                    send_sem=send_sems.at[off + k - 1],
                    recv_sem=recv_sems.at[off + k - 1],
                    device_id=(0,),
                    device_id_type=pl.DeviceIdType.MESH,
                )
                recv.wait_recv()
            rel = E - E_LOCAL * k
            for j in range(E_LOCAL):
                wq = commq_ref[k - 1][j * D:(j + 1) * D, :]
                sj = comms_ref[k - 1][j:j + 1, :].astype(jnp.bfloat16)
                wj = wq.astype(jnp.bfloat16) * sj
                acc = acc + jnp.dot(x_bf * coef_rot[:, rel + j:rel + j + 1],
                                    wj, preferred_element_type=jnp.float32)

        out_ref[...] = acc

        for i in range(2 * (N_DEV - 1)):
            send = pltpu.make_async_remote_copy(
                src_ref=sendq_ref if i < N_DEV - 1 else sends_ref,
                dst_ref=(commq_ref if i < N_DEV - 1 else comms_ref).at[0],
                send_sem=send_sems.at[i],
                recv_sem=recv_sems.at[i],
                device_id=(0,),
                device_id_type=pl.DeviceIdType.MESH,
            )
            send.wait_send()

    return pl.pallas_call(
        body,
        out_shape=jax.ShapeDtypeStruct((T, H), jnp.float32),
        in_specs=[pl.BlockSpec(memory_space=pltpu.VMEM)] * 5,
        out_specs=pl.BlockSpec(memory_space=pltpu.VMEM),
        scratch_shapes=[
            pltpu.VMEM((E_LOCAL * D, H), jnp.int8),
            pltpu.VMEM((E_LOCAL, H), jnp.float32),
            pltpu.VMEM((N_DEV - 1, E_LOCAL * D, H), jnp.int8),
            pltpu.VMEM((N_DEV - 1, E_LOCAL, H), jnp.float32),
            pltpu.SemaphoreType.DMA((2 * (N_DEV - 1),)),
            pltpu.SemaphoreType.DMA((2 * (N_DEV - 1),)),
        ],
        compiler_params=pltpu.CompilerParams(collective_id=0),
    )(x, router_W, route_idx, expert_W, shared_W)


# device time: 13840 ns/iter; 1.1341x vs baseline; 1.0132x over previous
import jax
import jax.numpy as jnp
from jax import lax
from jax.experimental import pallas as pl
from jax.experimental.pallas import tpu as pltpu

N_DEV = 8
E_LOCAL = 2
SROWS = E_LOCAL * 4


def kernel(x, router_W, route_idx, expert_W, shared_W):
    T, D = x.shape
    _, _, H = expert_W.shape
    E = N_DEV * E_LOCAL
    QR = E_LOCAL * D

    def body(x_ref, rw_ref, idx_ref, ew_ref, sw_ref, out_ref,
             sendbuf_ref, comm_ref, send_sems, recv_sems):
        my = lax.axis_index("i")

        barrier_sem = pltpu.get_barrier_semaphore()
        for k in range(1, N_DEV):
            pl.semaphore_signal(
                barrier_sem, inc=1,
                device_id=(lax.rem(my + k, N_DEV),),
                device_id_type=pl.DeviceIdType.MESH,
            )
        pl.semaphore_wait(barrier_sem, N_DEV - 1)

        scales = []
        for j in range(E_LOCAL):
            wj = ew_ref[j]
            mj = jnp.max(jnp.abs(wj), axis=0, keepdims=True)
            sinv = jnp.where(mj > 0, 127.0 / mj, 0.0)
            sendbuf_ref[j * D:(j + 1) * D, :] = jnp.round(
                wj * sinv).astype(jnp.int8)
            scales.append(mj * (1.0 / 127.0))
        sendbuf_ref[QR:QR + SROWS, :] = pltpu.bitcast(
            jnp.concatenate(scales, axis=0), jnp.int8)

        for k in range(1, N_DEV):
            rdma = pltpu.make_async_remote_copy(
                src_ref=sendbuf_ref,
                dst_ref=comm_ref.at[k - 1],
                send_sem=send_sems.at[k - 1],
                recv_sem=recv_sems.at[k - 1],
                device_id=(lax.rem(my + k, N_DEV),),
                device_id_type=pl.DeviceIdType.MESH,
            )
            rdma.start()

        x_bf = x_ref[...].astype(jnp.bfloat16)
        scores = jnp.dot(x_ref[...], rw_ref[...],
                         preferred_element_type=jnp.float32)
        s_max = jnp.max(scores, axis=-1, keepdims=True)
        p = jnp.exp(scores - s_max)
        probs = p / jnp.sum(p, axis=-1, keepdims=True)
        eidx = lax.broadcasted_iota(jnp.int32, (T, E), 1)
        coef = jnp.where(idx_ref[...] == eidx, probs, 0.0)
        coef_rot = pltpu.roll(
            coef.astype(jnp.bfloat16),
            jnp.mod(E - E_LOCAL * my, E), 1)

        acc = jnp.dot(x_bf, sw_ref[...].astype(jnp.bfloat16),
                      preferred_element_type=jnp.float32)

        ew_bf = ew_ref[...].astype(jnp.bfloat16)
        for j in range(E_LOCAL):
            acc = acc + jnp.dot(x_bf * coef_rot[:, j:j + 1], ew_bf[j],
                                preferred_element_type=jnp.float32)

        for k in range(1, N_DEV):
            recv = pltpu.make_async_remote_copy(
                src_ref=sendbuf_ref,
                dst_ref=comm_ref.at[k - 1],
                send_sem=send_sems.at[k - 1],
                recv_sem=recv_sems.at[k - 1],
                device_id=(0,),
                device_id_type=pl.DeviceIdType.MESH,
            )
            recv.wait_recv()
            sk = pltpu.bitcast(comm_ref[k - 1][QR:QR + SROWS, :],
                               jnp.float32)
            rel = E - E_LOCAL * k
            for j in range(E_LOCAL):
                wj = (comm_ref[k - 1][j * D:(j + 1) * D, :].astype(jnp.bfloat16)
                      * sk[j:j + 1, :].astype(jnp.bfloat16))
                acc = acc + jnp.dot(x_bf * coef_rot[:, rel + j:rel + j + 1],
                                    wj, preferred_element_type=jnp.float32)

        out_ref[...] = acc

        for k in range(1, N_DEV):
            send = pltpu.make_async_remote_copy(
                src_ref=sendbuf_ref,
                dst_ref=comm_ref.at[k - 1],
                send_sem=send_sems.at[k - 1],
                recv_sem=recv_sems.at[k - 1],
                device_id=(0,),
                device_id_type=pl.DeviceIdType.MESH,
            )
            send.wait_send()

    return pl.pallas_call(
        body,
        out_shape=jax.ShapeDtypeStruct((T, H), jnp.float32),
        in_specs=[pl.BlockSpec(memory_space=pltpu.VMEM)] * 5,
        out_specs=pl.BlockSpec(memory_space=pltpu.VMEM),
        scratch_shapes=[
            pltpu.VMEM((QR + SROWS, H), jnp.int8),
            pltpu.VMEM((N_DEV - 1, QR + SROWS, H), jnp.int8),
            pltpu.SemaphoreType.DMA((N_DEV - 1,)),
            pltpu.SemaphoreType.DMA((N_DEV - 1,)),
        ],
        compiler_params=pltpu.CompilerParams(collective_id=0),
    )(x, router_W, route_idx, expert_W, shared_W)


# device time: 13758 ns/iter; 1.1409x vs baseline; 1.0060x over previous
import jax
import jax.numpy as jnp
from jax import lax
from jax.experimental import pallas as pl
from jax.experimental.pallas import tpu as pltpu

N_DEV = 8
E_LOCAL = 2
SROWS = E_LOCAL * 4


def kernel(x, router_W, route_idx, expert_W, shared_W):
    T, D = x.shape
    _, _, H = expert_W.shape
    E = N_DEV * E_LOCAL
    QR = E_LOCAL * D

    def body(x_ref, rw_ref, idx_ref, ew_ref, sw_ref, out_ref,
             sendbuf_ref, comm_ref, send_sems, recv_sems):
        my = lax.axis_index("i")

        barrier_sem = pltpu.get_barrier_semaphore()
        for k in range(1, N_DEV):
            pl.semaphore_signal(
                barrier_sem, inc=1,
                device_id=(lax.rem(my + k, N_DEV),),
                device_id_type=pl.DeviceIdType.MESH,
            )

        scales = []
        for j in range(E_LOCAL):
            wj = ew_ref[j]
            mj = jnp.max(jnp.abs(wj), axis=0, keepdims=True)
            sinv = jnp.where(mj > 0, 127.0 / mj, 0.0)
            sendbuf_ref[j * D:(j + 1) * D, :] = jnp.round(
                wj * sinv).astype(jnp.int8)
            scales.append(mj * (1.0 / 127.0))
        sendbuf_ref[QR:QR + SROWS, :] = pltpu.bitcast(
            jnp.concatenate(scales, axis=0), jnp.int8)

        x_bf = x_ref[...].astype(jnp.bfloat16)
        scores = jnp.dot(x_ref[...], rw_ref[...],
                         preferred_element_type=jnp.float32)
        s_max = jnp.max(scores, axis=-1, keepdims=True)
        p = jnp.exp(scores - s_max)
        probs = p / jnp.sum(p, axis=-1, keepdims=True)
        eidx = lax.broadcasted_iota(jnp.int32, (T, E), 1)
        coef = jnp.where(idx_ref[...] == eidx, probs, 0.0)
        coef_rot = pltpu.roll(
            coef.astype(jnp.bfloat16),
            jnp.mod(E - E_LOCAL * my, E), 1)

        acc = jnp.dot(x_bf, sw_ref[...].astype(jnp.bfloat16),
                      preferred_element_type=jnp.float32)

        ew_bf = ew_ref[...].astype(jnp.bfloat16)
        for j in range(E_LOCAL):
            acc = acc + jnp.dot(x_bf * coef_rot[:, j:j + 1], ew_bf[j],
                                preferred_element_type=jnp.float32)

        pl.semaphore_wait(barrier_sem, N_DEV - 1)

        for k in range(1, N_DEV):
            rdma = pltpu.make_async_remote_copy(
                src_ref=sendbuf_ref,
                dst_ref=comm_ref.at[k - 1],
                send_sem=send_sems.at[k - 1],
                recv_sem=recv_sems.at[k - 1],
                device_id=(lax.rem(my + k, N_DEV),),
                device_id_type=pl.DeviceIdType.MESH,
            )
            rdma.start()

        for k in range(1, N_DEV):
            recv = pltpu.make_async_remote_copy(
                src_ref=sendbuf_ref,
                dst_ref=comm_ref.at[k - 1],
                send_sem=send_sems.at[k - 1],
                recv_sem=recv_sems.at[k - 1],
                device_id=(0,),
                device_id_type=pl.DeviceIdType.MESH,
            )
            recv.wait_recv()
            sk = pltpu.bitcast(comm_ref[k - 1][QR:QR + SROWS, :],
                               jnp.float32)
            rel = E - E_LOCAL * k
            for j in range(E_LOCAL):
                wj = (comm_ref[k - 1][j * D:(j + 1) * D, :].astype(jnp.bfloat16)
                      * sk[j:j + 1, :].astype(jnp.bfloat16))
                acc = acc + jnp.dot(x_bf * coef_rot[:, rel + j:rel + j + 1],
                                    wj, preferred_element_type=jnp.float32)

        out_ref[...] = acc

        for k in range(1, N_DEV):
            send = pltpu.make_async_remote_copy(
                src_ref=sendbuf_ref,
                dst_ref=comm_ref.at[k - 1],
                send_sem=send_sems.at[k - 1],
                recv_sem=recv_sems.at[k - 1],
                device_id=(0,),
                device_id_type=pl.DeviceIdType.MESH,
            )
            send.wait_send()

    return pl.pallas_call(
        body,
        out_shape=jax.ShapeDtypeStruct((T, H), jnp.float32),
        in_specs=[pl.BlockSpec(memory_space=pltpu.VMEM)] * 5,
        out_specs=pl.BlockSpec(memory_space=pltpu.VMEM),
        scratch_shapes=[
            pltpu.VMEM((QR + SROWS, H), jnp.int8),
            pltpu.VMEM((N_DEV - 1, QR + SROWS, H), jnp.int8),
            pltpu.SemaphoreType.DMA((N_DEV - 1,)),
            pltpu.SemaphoreType.DMA((N_DEV - 1,)),
        ],
        compiler_params=pltpu.CompilerParams(collective_id=0),
    )(x, router_W, route_idx, expert_W, shared_W)
